# Initial kernel scaffold; baseline (speedup 1.0000x reference)
import jax
import jax.numpy as jnp
from jax.experimental import pallas as pl


def kernel(x, edge_index, batch, W1, b1, W2, b2, W3, b3, W4, b4, W5, b5, lin1_W, lin1_b, lin2_W, lin2_b):
    raise NotImplementedError("write your pallas kernel here")


# speedup vs baseline: 38.2893x; 38.2893x over previous
import functools

import jax
import jax.numpy as jnp
from jax import lax
from jax.experimental import pallas as pl
from jax.experimental.pallas import tpu as pltpu
from jax.experimental.pallas import tpu_sc as plsc

N = 100000
E = 3200000
H = 16
G = 64
SLOPE = (1.0 / 8.0 + 1.0 / 3.0) / 2.0

NC = 2
NS = 16
NW = NC * NS
LANE = 128

EROWS = 25088
EPAD = EROWS * LANE - E
RPT = EROWS // NW
CH = 8
NCHUNK = RPT // CH

NACC = 100096
RACC = NACC // NS
ZROWS = RACC // 8
DUMMY = N

TCB = 4000
TCG = N // TCB

_f32 = jnp.float32
_i32 = jnp.int32

_sc_mesh = plsc.VectorSubcoreMesh(core_axis_name="c", subcore_axis_name="s")
_sc_params = pltpu.CompilerParams(use_tc_tiling_on_sc=False)


def _deg_body(dst_hbm, out_hbm, dstv, onesv, zbuf, degsh):
    c = lax.axis_index("c")
    s = lax.axis_index("s")
    wid = s * NC + c

    def _zero(i, carry):
        zbuf[pl.ds(i * 16, 16)] = jnp.zeros((16,), _f32)
        return carry

    lax.fori_loop(0, RACC // 16, _zero, 0)

    def _ones(i, carry):
        onesv[pl.ds(i * 16, 16)] = jnp.ones((16,), _f32)
        return carry

    lax.fori_loop(0, LANE // 16, _ones, 0)

    pltpu.sync_copy(zbuf, degsh.at[pl.ds(s * RACC, RACC)])
    plsc.subcore_barrier()

    def _chunk(i, carry):
        row0 = wid * RPT + i * CH
        pltpu.sync_copy(dst_hbm.at[pl.ds(row0, CH)], dstv)
        for j in range(CH):
            pltpu.sync_copy(onesv, degsh.at[dstv.at[j]], add=True)
        return carry

    lax.fori_loop(0, NCHUNK, _chunk, 0)
    plsc.subcore_barrier()
    pltpu.sync_copy(degsh.at[pl.ds(s * RACC, RACC)],
                    out_hbm.at[c, pl.ds(s * RACC, RACC)])


_deg_kernel = pl.kernel(
    _deg_body,
    out_type=jax.ShapeDtypeStruct((NC, NACC), _f32),
    mesh=_sc_mesh,
    scratch_types=[
        pltpu.VMEM((CH, LANE), _i32),
        pltpu.VMEM((LANE,), _f32),
        pltpu.VMEM((RACC,), _f32),
        pltpu.VMEM_SHARED((NACC,), _f32),
    ],
    compiler_params=_sc_params,
)


def _msg_body(z_hbm, src_hbm, dst_hbm, out_hbm, srcv, dstv, rowsv, zbuf,
              accsh, sem):
    c = lax.axis_index("c")
    s = lax.axis_index("s")
    wid = s * NC + c

    def _zero(i, carry):
        zbuf[i] = jnp.zeros((16,), _f32)
        return carry

    lax.fori_loop(0, ZROWS, _zero, 0)
    for k in range(RACC // ZROWS):
        pltpu.sync_copy(zbuf, accsh.at[pl.ds(s * RACC + k * ZROWS, ZROWS)])
    plsc.subcore_barrier()

    def _chunk(i, carry):
        row0 = wid * RPT + i * CH
        pltpu.sync_copy(src_hbm.at[pl.ds(row0, CH)], srcv)
        pltpu.sync_copy(dst_hbm.at[pl.ds(row0, CH)], dstv)
        cps = [pltpu.async_copy(z_hbm.at[srcv.at[j]], rowsv.at[j], sem)
               for j in range(CH)]
        for cp in cps:
            cp.wait()
        for j in range(CH):
            pltpu.sync_copy(rowsv.at[j], accsh.at[dstv.at[j]], add=True)
        return carry

    lax.fori_loop(0, NCHUNK, _chunk, 0)
    plsc.subcore_barrier()
    pltpu.sync_copy(accsh.at[pl.ds(s * RACC, RACC)],
                    out_hbm.at[c, pl.ds(s * RACC, RACC)])


_msg_kernel = pl.kernel(
    _msg_body,
    out_type=jax.ShapeDtypeStruct((NC, NACC, H), _f32),
    mesh=_sc_mesh,
    scratch_types=[
        pltpu.VMEM((CH, LANE), _i32),
        pltpu.VMEM((CH, LANE), _i32),
        pltpu.VMEM((CH, LANE, H), _f32),
        pltpu.VMEM((ZROWS, H), _f32),
        pltpu.VMEM_SHARED((NACC, H), _f32),
        pltpu.SemaphoreType.DMA,
    ],
    compiler_params=_sc_params,
)


def _k0_body(deg_ref, x_ref, w_ref, dinv_ref, z_ref):
    deg = deg_ref[0] + deg_ref[1] + 1.0
    dinv = lax.rsqrt(deg)
    dinv_ref[...] = dinv
    y = jnp.dot(x_ref[...], w_ref[...], preferred_element_type=_f32)
    z_ref[...] = y * dinv


_k0 = pl.pallas_call(
    _k0_body,
    grid=(TCG,),
    in_specs=[
        pl.BlockSpec((NC, TCB, 1), lambda i: (0, i, 0)),
        pl.BlockSpec((TCB, 4), lambda i: (i, 0)),
        pl.BlockSpec((4, H), lambda i: (0, 0)),
    ],
    out_specs=[
        pl.BlockSpec((TCB, 1), lambda i: (i, 0)),
        pl.BlockSpec((TCB, H), lambda i: (i, 0)),
    ],
    out_shape=[
        jax.ShapeDtypeStruct((N, 1), _f32),
        jax.ShapeDtypeStruct((N, H), _f32),
    ],
)


def _layer_body(acc_ref, z_ref, dinv_ref, b_ref, w_ref, out_ref):
    dinv = dinv_ref[...]
    pre = (acc_ref[0] + acc_ref[1] + z_ref[...]) * dinv + b_ref[...]
    h = jnp.maximum(pre, 0.0)
    out_ref[...] = jnp.dot(h, w_ref[...], preferred_element_type=_f32) * dinv


_layer = pl.pallas_call(
    _layer_body,
    grid=(TCG,),
    in_specs=[
        pl.BlockSpec((NC, TCB, H), lambda i: (0, i, 0)),
        pl.BlockSpec((TCB, H), lambda i: (i, 0)),
        pl.BlockSpec((TCB, 1), lambda i: (i, 0)),
        pl.BlockSpec((1, H), lambda i: (0, 0)),
        pl.BlockSpec((H, H), lambda i: (0, 0)),
    ],
    out_specs=pl.BlockSpec((TCB, H), lambda i: (i, 0)),
    out_shape=jax.ShapeDtypeStruct((N, H), _f32),
)


def _pool_body(acc_ref, z_ref, dinv_ref, b_ref, batch_ref, l1w_ref, l1b_ref,
               l2w_ref, l2b_ref, out_ref, sums_ref):
    i = pl.program_id(0)

    @pl.when(i == 0)
    def _():
        sums_ref[...] = jnp.zeros_like(sums_ref)

    dinv = dinv_ref[...]
    pre = (acc_ref[0] + acc_ref[1] + z_ref[...]) * dinv + b_ref[...]
    h = jnp.maximum(pre, 0.0)
    hh = jnp.concatenate([h, jnp.ones((TCB, 1), _f32)], axis=1)
    seg = (batch_ref[...] == lax.broadcasted_iota(_i32, (1, G), 1))
    seg = seg.astype(_f32)
    sums_ref[...] += lax.dot_general(seg, hh, (((0,), (0,)), ((), ())),
                                     preferred_element_type=_f32)

    @pl.when(i == TCG - 1)
    def _():
        tot = sums_ref[...]
        mean = tot[:, :H] / jnp.maximum(tot[:, H:H + 1], 1.0)
        g = jnp.dot(mean, l1w_ref[...], preferred_element_type=_f32)
        g = g + l1b_ref[...]
        g = jnp.where(g >= 0, g, SLOPE * g)
        o = jnp.dot(g, l2w_ref[...], preferred_element_type=_f32)
        o = o + l2b_ref[...]
        out_ref[...] = jnp.where(o >= 0, o, SLOPE * o)


_pool = pl.pallas_call(
    _pool_body,
    grid=(TCG,),
    in_specs=[
        pl.BlockSpec((NC, TCB, H), lambda i: (0, i, 0)),
        pl.BlockSpec((TCB, H), lambda i: (i, 0)),
        pl.BlockSpec((TCB, 1), lambda i: (i, 0)),
        pl.BlockSpec((1, H), lambda i: (0, 0)),
        pl.BlockSpec((TCB, 1), lambda i: (i, 0)),
        pl.BlockSpec((H, H), lambda i: (0, 0)),
        pl.BlockSpec((1, H), lambda i: (0, 0)),
        pl.BlockSpec((H, 1), lambda i: (0, 0)),
        pl.BlockSpec((1, 1), lambda i: (0, 0)),
    ],
    out_specs=pl.BlockSpec((G, 1), lambda i: (0, 0)),
    out_shape=jax.ShapeDtypeStruct((G, 1), _f32),
    scratch_shapes=[pltpu.VMEM((G, H + 1), _f32)],
)


def kernel(x, edge_index, batch, W1, b1, W2, b2, W3, b3, W4, b4, W5, b5,
           lin1_W, lin1_b, lin2_W, lin2_b):
    src = edge_index[0].astype(_i32)
    dst = edge_index[1].astype(_i32)
    src2d = jnp.concatenate([src, jnp.zeros((EPAD,), _i32)]).reshape(EROWS, LANE)
    dst2d = jnp.concatenate([dst, jnp.full((EPAD,), DUMMY, _i32)]).reshape(EROWS, LANE)

    x_p = x
    batch_p = batch.astype(_i32).reshape(N, 1)

    deg = _deg_kernel(dst2d).reshape(NC, NACC, 1)
    dinv, z = _k0(deg, x_p, W1)

    bs = [b1, b2, b3, b4]
    ws = [W2, W3, W4, W5]
    for b, w in zip(bs, ws):
        acc = _msg_kernel(z, src2d, dst2d)
        z = _layer(acc, z, dinv, b.reshape(1, H), w)

    acc = _msg_kernel(z, src2d, dst2d)
    out = _pool(acc, z, dinv, b5.reshape(1, H), batch_p,
                lin1_W, lin1_b.reshape(1, H), lin2_W, lin2_b.reshape(1, 1))
    return out


# speedup vs baseline: 46.2589x; 1.2081x over previous
import functools

import jax
import jax.numpy as jnp
from jax import lax
from jax.experimental import pallas as pl
from jax.experimental.pallas import tpu as pltpu
from jax.experimental.pallas import tpu_sc as plsc

N = 100000
E = 3200000
H = 16
G = 64
SLOPE = (1.0 / 8.0 + 1.0 / 3.0) / 2.0

NC = 2
NS = 16
NW = NC * NS
LANE = 128

EROWS = 25088
EPAD = EROWS * LANE - E
RPT = EROWS // NW
CH = 4
NCHUNK = RPT // CH

NACC = 100096
RACC = NACC // NS
ZROWS = RACC // 8
DUMMY = N

TCB = 4000
TCG = N // TCB

_f32 = jnp.float32
_i32 = jnp.int32

_sc_mesh = plsc.VectorSubcoreMesh(core_axis_name="c", subcore_axis_name="s")
_sc_params = pltpu.CompilerParams(use_tc_tiling_on_sc=False)


def _deg_body(dst_hbm, out_hbm, dstv, onesv, zbuf, degsh, ssem):
    c = lax.axis_index("c")
    s = lax.axis_index("s")
    wid = s * NC + c

    def _zero(i, carry):
        zbuf[pl.ds(i * 16, 16)] = jnp.zeros((16,), _f32)
        return carry

    lax.fori_loop(0, RACC // 16, _zero, 0)

    def _ones(i, carry):
        onesv[pl.ds(i * 16, 16)] = jnp.ones((16,), _f32)
        return carry

    lax.fori_loop(0, LANE // 16, _ones, 0)

    pltpu.sync_copy(zbuf, degsh.at[pl.ds(s * RACC, RACC)])
    plsc.subcore_barrier()

    def load_idx(slot, chunk):
        row0 = wid * RPT + chunk * CH
        pltpu.sync_copy(dst_hbm.at[pl.ds(row0, CH)], dstv.at[slot])

    def fire_sc(slot):
        for j in range(CH):
            pltpu.async_copy(onesv, degsh.at[dstv.at[slot, j]], ssem,
                             add=True)

    def wait_sc(slot):
        for j in range(CH):
            pltpu.make_async_copy(onesv, degsh.at[dstv.at[slot, j]],
                                  ssem).wait()

    load_idx(0, 0)
    fire_sc(0)
    load_idx(1, 1)
    fire_sc(1)

    def _chunk(i, carry):
        p = lax.rem(i, 2)
        wait_sc(p)
        load_idx(p, i)
        fire_sc(p)
        return carry

    lax.fori_loop(2, NCHUNK, _chunk, 0)
    wait_sc(NCHUNK % 2)
    wait_sc(1 - NCHUNK % 2)
    plsc.subcore_barrier()
    pltpu.sync_copy(degsh.at[pl.ds(s * RACC, RACC)],
                    out_hbm.at[c, pl.ds(s * RACC, RACC)])


_deg_kernel = pl.kernel(
    _deg_body,
    out_type=jax.ShapeDtypeStruct((NC, NACC), _f32),
    mesh=_sc_mesh,
    scratch_types=[
        pltpu.VMEM((2, CH, LANE), _i32),
        pltpu.VMEM((LANE,), _f32),
        pltpu.VMEM((RACC,), _f32),
        pltpu.VMEM_SHARED((NACC,), _f32),
        pltpu.SemaphoreType.DMA,
    ],
    compiler_params=_sc_params,
)


def _msg_body(z_hbm, src_hbm, dst_hbm, out_hbm, srcv, dstv, rowsv, zbuf,
              accsh, gsem, ssem):
    c = lax.axis_index("c")
    s = lax.axis_index("s")
    wid = s * NC + c

    def _zero(i, carry):
        zbuf[i] = jnp.zeros((16,), _f32)
        return carry

    lax.fori_loop(0, ZROWS, _zero, 0)
    for k in range(RACC // ZROWS):
        pltpu.sync_copy(zbuf, accsh.at[pl.ds(s * RACC + k * ZROWS, ZROWS)])
    plsc.subcore_barrier()

    def load_idx(slot, chunk):
        row0 = wid * RPT + chunk * CH
        pltpu.sync_copy(src_hbm.at[pl.ds(row0, CH)], srcv.at[slot])
        pltpu.sync_copy(dst_hbm.at[pl.ds(row0, CH)], dstv.at[slot])

    def fire_g(slot):
        for j in range(CH):
            pltpu.async_copy(z_hbm.at[srcv.at[slot, j]], rowsv.at[slot, j],
                             gsem)

    def wait_g(slot):
        for j in range(CH):
            pltpu.make_async_copy(z_hbm.at[srcv.at[slot, j]],
                                  rowsv.at[slot, j], gsem).wait()

    def fire_s(slot):
        for j in range(CH):
            pltpu.async_copy(rowsv.at[slot, j], accsh.at[dstv.at[slot, j]],
                             ssem, add=True)

    def wait_s(slot):
        for j in range(CH):
            pltpu.make_async_copy(rowsv.at[slot, j],
                                  accsh.at[dstv.at[slot, j]], ssem).wait()

    load_idx(0, 0)
    fire_g(0)
    load_idx(1, 1)
    wait_g(0)
    fire_s(0)
    fire_g(1)

    def _chunk(i, carry):
        p = lax.rem(i, 2)
        q = 1 - p
        wait_s(q)
        load_idx(q, i + 1)
        wait_g(p)
        fire_s(p)
        fire_g(q)
        return carry

    lax.fori_loop(1, NCHUNK - 1, _chunk, 0)
    pl_last = (NCHUNK - 1) % 2
    wait_s(1 - pl_last)
    wait_g(pl_last)
    fire_s(pl_last)
    wait_s(pl_last)
    plsc.subcore_barrier()
    pltpu.sync_copy(accsh.at[pl.ds(s * RACC, RACC)],
                    out_hbm.at[c, pl.ds(s * RACC, RACC)])


_msg_kernel = pl.kernel(
    _msg_body,
    out_type=jax.ShapeDtypeStruct((NC, NACC, H), _f32),
    mesh=_sc_mesh,
    scratch_types=[
        pltpu.VMEM((2, CH, LANE), _i32),
        pltpu.VMEM((2, CH, LANE), _i32),
        pltpu.VMEM((2, CH, LANE, H), _f32),
        pltpu.VMEM((ZROWS, H), _f32),
        pltpu.VMEM_SHARED((NACC, H), _f32),
        pltpu.SemaphoreType.DMA,
        pltpu.SemaphoreType.DMA,
    ],
    compiler_params=_sc_params,
)


def _k0_body(deg_ref, x_ref, w_ref, dinv_ref, z_ref):
    deg = deg_ref[0] + deg_ref[1] + 1.0
    dinv = lax.rsqrt(deg)
    dinv_ref[...] = dinv
    y = jnp.dot(x_ref[...], w_ref[...], preferred_element_type=_f32)
    z_ref[...] = y * dinv


_k0 = pl.pallas_call(
    _k0_body,
    grid=(TCG,),
    in_specs=[
        pl.BlockSpec((NC, TCB, 1), lambda i: (0, i, 0)),
        pl.BlockSpec((TCB, 4), lambda i: (i, 0)),
        pl.BlockSpec((4, H), lambda i: (0, 0)),
    ],
    out_specs=[
        pl.BlockSpec((TCB, 1), lambda i: (i, 0)),
        pl.BlockSpec((TCB, H), lambda i: (i, 0)),
    ],
    out_shape=[
        jax.ShapeDtypeStruct((N, 1), _f32),
        jax.ShapeDtypeStruct((N, H), _f32),
    ],
)


def _layer_body(acc_ref, z_ref, dinv_ref, b_ref, w_ref, out_ref):
    dinv = dinv_ref[...]
    pre = (acc_ref[0] + acc_ref[1] + z_ref[...]) * dinv + b_ref[...]
    h = jnp.maximum(pre, 0.0)
    out_ref[...] = jnp.dot(h, w_ref[...], preferred_element_type=_f32) * dinv


_layer = pl.pallas_call(
    _layer_body,
    grid=(TCG,),
    in_specs=[
        pl.BlockSpec((NC, TCB, H), lambda i: (0, i, 0)),
        pl.BlockSpec((TCB, H), lambda i: (i, 0)),
        pl.BlockSpec((TCB, 1), lambda i: (i, 0)),
        pl.BlockSpec((1, H), lambda i: (0, 0)),
        pl.BlockSpec((H, H), lambda i: (0, 0)),
    ],
    out_specs=pl.BlockSpec((TCB, H), lambda i: (i, 0)),
    out_shape=jax.ShapeDtypeStruct((N, H), _f32),
)


def _pool_body(acc_ref, z_ref, dinv_ref, b_ref, batch_ref, l1w_ref, l1b_ref,
               l2w_ref, l2b_ref, out_ref, sums_ref):
    i = pl.program_id(0)

    @pl.when(i == 0)
    def _():
        sums_ref[...] = jnp.zeros_like(sums_ref)

    dinv = dinv_ref[...]
    pre = (acc_ref[0] + acc_ref[1] + z_ref[...]) * dinv + b_ref[...]
    h = jnp.maximum(pre, 0.0)
    hh = jnp.concatenate([h, jnp.ones((TCB, 1), _f32)], axis=1)
    seg = (batch_ref[...] == lax.broadcasted_iota(_i32, (1, G), 1))
    seg = seg.astype(_f32)
    sums_ref[...] += lax.dot_general(seg, hh, (((0,), (0,)), ((), ())),
                                     preferred_element_type=_f32)

    @pl.when(i == TCG - 1)
    def _():
        tot = sums_ref[...]
        mean = tot[:, :H] / jnp.maximum(tot[:, H:H + 1], 1.0)
        g = jnp.dot(mean, l1w_ref[...], preferred_element_type=_f32)
        g = g + l1b_ref[...]
        g = jnp.where(g >= 0, g, SLOPE * g)
        o = jnp.dot(g, l2w_ref[...], preferred_element_type=_f32)
        o = o + l2b_ref[...]
        out_ref[...] = jnp.where(o >= 0, o, SLOPE * o)


_pool = pl.pallas_call(
    _pool_body,
    grid=(TCG,),
    in_specs=[
        pl.BlockSpec((NC, TCB, H), lambda i: (0, i, 0)),
        pl.BlockSpec((TCB, H), lambda i: (i, 0)),
        pl.BlockSpec((TCB, 1), lambda i: (i, 0)),
        pl.BlockSpec((1, H), lambda i: (0, 0)),
        pl.BlockSpec((TCB, 1), lambda i: (i, 0)),
        pl.BlockSpec((H, H), lambda i: (0, 0)),
        pl.BlockSpec((1, H), lambda i: (0, 0)),
        pl.BlockSpec((H, 1), lambda i: (0, 0)),
        pl.BlockSpec((1, 1), lambda i: (0, 0)),
    ],
    out_specs=pl.BlockSpec((G, 1), lambda i: (0, 0)),
    out_shape=jax.ShapeDtypeStruct((G, 1), _f32),
    scratch_shapes=[pltpu.VMEM((G, H + 1), _f32)],
)


def kernel(x, edge_index, batch, W1, b1, W2, b2, W3, b3, W4, b4, W5, b5,
           lin1_W, lin1_b, lin2_W, lin2_b):
    src = edge_index[0].astype(_i32)
    dst = edge_index[1].astype(_i32)
    src2d = jnp.concatenate([src, jnp.zeros((EPAD,), _i32)]).reshape(EROWS, LANE)
    dst2d = jnp.concatenate([dst, jnp.full((EPAD,), DUMMY, _i32)]).reshape(EROWS, LANE)

    x_p = x
    batch_p = batch.astype(_i32).reshape(N, 1)

    deg = _deg_kernel(dst2d).reshape(NC, NACC, 1)
    dinv, z = _k0(deg, x_p, W1)

    bs = [b1, b2, b3, b4]
    ws = [W2, W3, W4, W5]
    for b, w in zip(bs, ws):
        acc = _msg_kernel(z, src2d, dst2d)
        z = _layer(acc, z, dinv, b.reshape(1, H), w)

    acc = _msg_kernel(z, src2d, dst2d)
    out = _pool(acc, z, dinv, b5.reshape(1, H), batch_p,
                lin1_W, lin1_b.reshape(1, H), lin2_W, lin2_b.reshape(1, 1))
    return out


# speedup vs baseline: 61.2369x; 1.3238x over previous
import jax
import jax.numpy as jnp
from jax import lax
from jax.experimental import pallas as pl
from jax.experimental.pallas import tpu as pltpu
from jax.experimental.pallas import tpu_sc as plsc

N = 100000
E = 3200000
H = 16
G = 64
SLOPE = (1.0 / 8.0 + 1.0 / 3.0) / 2.0

NC = 2
NS = 16
NW = NC * NS
LANE = 128

EROWS = 25088
EPAD = EROWS * LANE - E
RPT = EROWS // NW
CH = 4
NCHUNK = RPT // CH
RPT2 = EROWS // NS
NCHUNK2 = RPT2 // CH

NACC = 100096
RACC = NACC // NS
ZROWS = RACC // 8
DUMMY = N
NHALF = NACC // NC
NT = NACC // NW
NV = 196

NP = NACC * H // 128
PB = 368
PGRID = NP // PB

_f32 = jnp.float32
_i32 = jnp.int32

_sc_mesh = plsc.VectorSubcoreMesh(core_axis_name="c", subcore_axis_name="s")
_sc_params = pltpu.CompilerParams(use_tc_tiling_on_sc=False,
                                  needs_layout_passes=False)


def _prep_body(dst_hbm, out_hbm, dstv, onesv, dbuf, obuf, gb, degsh, ssem):
    c = lax.axis_index("c")
    s = lax.axis_index("s")

    def _zero(i, carry):
        obuf[pl.ds(i * 16, 16)] = jnp.zeros((16,), _f32)
        return carry

    lax.fori_loop(0, RACC // 16, _zero, 0)

    def _ones(i, carry):
        onesv[pl.ds(i * 16, 16)] = jnp.ones((16,), _f32)
        return carry

    lax.fori_loop(0, LANE // 16, _ones, 0)

    pltpu.sync_copy(obuf.at[pl.ds(0, RACC)], degsh.at[pl.ds(s * RACC, RACC)])
    plsc.subcore_barrier()

    def load_idx(slot, chunk):
        row0 = s * RPT2 + chunk * CH
        pltpu.sync_copy(dst_hbm.at[pl.ds(row0, CH)], dstv.at[slot])

    def fire_sc(slot):
        for j in range(CH):
            pltpu.async_copy(onesv, degsh.at[dstv.at[slot, j]], ssem,
                             add=True)

    def wait_sc(slot):
        for j in range(CH):
            pltpu.make_async_copy(onesv, degsh.at[dstv.at[slot, j]],
                                  ssem).wait()

    load_idx(0, 0)
    fire_sc(0)
    load_idx(1, 1)
    fire_sc(1)

    def _chunk(i, carry):
        p = lax.rem(i, 2)
        wait_sc(p)
        load_idx(p, i)
        fire_sc(p)
        return carry

    lax.fori_loop(2, NCHUNK2, _chunk, 0)
    wait_sc(NCHUNK2 % 2)
    wait_sc(1 - NCHUNK2 % 2)
    plsc.subcore_barrier()

    base = c * NHALF + s * NT
    pltpu.sync_copy(degsh.at[pl.ds(base, NT)], dbuf.at[pl.ds(0, NT)])

    def _rsqrt(v, carry):
        d = dbuf[pl.ds(v * 16, 16)] + 1.0
        ii = plsc.bitcast(d, _i32)
        gi = jnp.full((16,), 0x5F3759DF, _i32) - (ii >> 1)
        g = plsc.bitcast(gi, _f32)
        hx = 0.5 * d
        g = g * (1.5 - hx * g * g)
        g = g * (1.5 - hx * g * g)
        g = g * (1.5 - hx * g * g)
        gb[...] = g
        for a in range(16):
            vec = plsc.load_gather(gb, [jnp.full((16,), a, _i32)])
            obuf[pl.ds(v * 256 + a * 16, 16)] = vec
        return carry

    lax.fori_loop(0, NV, _rsqrt, 0)
    pltpu.sync_copy(obuf.at[pl.ds(0, NT * 16)],
                    out_hbm.at[pl.ds(base * 16, NT * 16)])


_prep_kernel = pl.kernel(
    _prep_body,
    out_type=jax.ShapeDtypeStruct((NACC * H,), _f32),
    mesh=_sc_mesh,
    scratch_types=[
        pltpu.VMEM((2, CH, LANE), _i32),
        pltpu.VMEM((LANE,), _f32),
        pltpu.VMEM((NT + 8,), _f32),
        pltpu.VMEM((NT * 16,), _f32),
        pltpu.VMEM((16,), _f32),
        pltpu.VMEM_SHARED((NACC,), _f32),
        pltpu.SemaphoreType.DMA,
    ],
    compiler_params=_sc_params,
)


def _msg_body(z_hbm, src_hbm, dst_hbm, out_hbm, srcv, dstv, rowsv, zbuf,
              accsh, gsem, ssem):
    c = lax.axis_index("c")
    s = lax.axis_index("s")
    wid = s * NC + c

    def _zero(i, carry):
        zbuf[i] = jnp.zeros((16,), _f32)
        return carry

    lax.fori_loop(0, ZROWS, _zero, 0)
    for k in range(RACC // ZROWS):
        pltpu.sync_copy(zbuf, accsh.at[pl.ds(s * RACC + k * ZROWS, ZROWS)])
    plsc.subcore_barrier()

    def load_idx(slot, chunk):
        row0 = wid * RPT + chunk * CH
        pltpu.sync_copy(src_hbm.at[pl.ds(row0, CH)], srcv.at[slot])
        pltpu.sync_copy(dst_hbm.at[pl.ds(row0, CH)], dstv.at[slot])

    def fire_g(slot):
        for j in range(CH):
            pltpu.async_copy(z_hbm.at[srcv.at[slot, j]], rowsv.at[slot, j],
                             gsem)

    def wait_g(slot):
        for j in range(CH):
            pltpu.make_async_copy(z_hbm.at[srcv.at[slot, j]],
                                  rowsv.at[slot, j], gsem).wait()

    def fire_s(slot):
        for j in range(CH):
            pltpu.async_copy(rowsv.at[slot, j], accsh.at[dstv.at[slot, j]],
                             ssem, add=True)

    def wait_s(slot):
        for j in range(CH):
            pltpu.make_async_copy(rowsv.at[slot, j],
                                  accsh.at[dstv.at[slot, j]], ssem).wait()

    load_idx(0, 0)
    fire_g(0)
    load_idx(1, 1)
    wait_g(0)
    fire_s(0)
    fire_g(1)

    def _chunk(i, carry):
        p = lax.rem(i, 2)
        q = 1 - p
        wait_s(q)
        load_idx(q, i + 1)
        wait_g(p)
        fire_s(p)
        fire_g(q)
        return carry

    lax.fori_loop(1, NCHUNK - 1, _chunk, 0)
    pl_last = (NCHUNK - 1) % 2
    wait_s(1 - pl_last)
    wait_g(pl_last)
    fire_s(pl_last)
    wait_s(pl_last)
    plsc.subcore_barrier()
    pltpu.sync_copy(accsh.at[pl.ds(s * RACC, RACC)],
                    out_hbm.at[c, pl.ds(s * RACC, RACC)])


_msg_kernel = pl.kernel(
    _msg_body,
    out_type=jax.ShapeDtypeStruct((NC, NACC, H), _f32),
    mesh=_sc_mesh,
    scratch_types=[
        pltpu.VMEM((2, CH, LANE), _i32),
        pltpu.VMEM((2, CH, LANE), _i32),
        pltpu.VMEM((2, CH, LANE, H), _f32),
        pltpu.VMEM((ZROWS, H), _f32),
        pltpu.VMEM_SHARED((NACC, H), _f32),
        pltpu.SemaphoreType.DMA,
        pltpu.SemaphoreType.DMA,
    ],
    compiler_params=_sc_params,
)


def _zinit_body(x_ref, dinvx_ref, w_ref, z_ref):
    y = jnp.dot(x_ref[...], w_ref[...], preferred_element_type=_f32)
    z_ref[...] = y * dinvx_ref[...]


_zinit = pl.pallas_call(
    _zinit_body,
    grid=(PGRID,),
    in_specs=[
        pl.BlockSpec((PB, 128), lambda i: (i, 0)),
        pl.BlockSpec((PB, 128), lambda i: (i, 0)),
        pl.BlockSpec((128, 128), lambda i: (0, 0)),
    ],
    out_specs=pl.BlockSpec((PB, 128), lambda i: (i, 0)),
    out_shape=jax.ShapeDtypeStruct((NP, 128), _f32),
)


def _layer_body(acc_ref, z_ref, dinvx_ref, b8_ref, w8_ref, out_ref):
    dinvx = dinvx_ref[...]
    pre = (acc_ref[0] + acc_ref[1] + z_ref[...]) * dinvx + b8_ref[...]
    h = jnp.maximum(pre, 0.0)
    out_ref[...] = jnp.dot(h, w8_ref[...],
                           preferred_element_type=_f32) * dinvx


_layer = pl.pallas_call(
    _layer_body,
    grid=(PGRID,),
    in_specs=[
        pl.BlockSpec((NC, PB, 128), lambda i: (0, i, 0)),
        pl.BlockSpec((PB, 128), lambda i: (i, 0)),
        pl.BlockSpec((PB, 128), lambda i: (i, 0)),
        pl.BlockSpec((1, 128), lambda i: (0, 0)),
        pl.BlockSpec((128, 128), lambda i: (0, 0)),
    ],
    out_specs=pl.BlockSpec((PB, 128), lambda i: (i, 0)),
    out_shape=jax.ShapeDtypeStruct((NP, 128), _f32),
)


def _pool_body(acc_ref, z_ref, dinvx_ref, b8_ref, batch_ref, l1w_ref,
               l1b_ref, l2w_ref, l2b_ref, out_ref, sums_ref):
    i = pl.program_id(0)

    @pl.when(i == 0)
    def _():
        sums_ref[...] = jnp.zeros_like(sums_ref)

    pre = (acc_ref[0] + acc_ref[1] + z_ref[...]) * dinvx_ref[...]
    h_pk = jnp.maximum(pre + b8_ref[...], 0.0)
    iota_g = lax.broadcasted_iota(_i32, (1, G), 1)
    acc = jnp.zeros((G, H + 1), _f32)
    for a in range(8):
        ha = h_pk[:, a * H:(a + 1) * H]
        hha = jnp.concatenate([ha, jnp.ones((PB, 1), _f32)], axis=1)
        sega = (batch_ref[:, a:a + 1] == iota_g).astype(_f32)
        acc = acc + lax.dot_general(sega, hha, (((0,), (0,)), ((), ())),
                                    preferred_element_type=_f32)
    sums_ref[...] += acc

    @pl.when(i == PGRID - 1)
    def _():
        tot = sums_ref[...]
        mean = tot[:, :H] / jnp.maximum(tot[:, H:H + 1], 1.0)
        g = jnp.dot(mean, l1w_ref[...], preferred_element_type=_f32)
        g = g + l1b_ref[...]
        g = jnp.where(g >= 0, g, SLOPE * g)
        o = jnp.dot(g, l2w_ref[...], preferred_element_type=_f32)
        o = o + l2b_ref[...]
        out_ref[...] = jnp.where(o >= 0, o, SLOPE * o)


_pool = pl.pallas_call(
    _pool_body,
    grid=(PGRID,),
    in_specs=[
        pl.BlockSpec((NC, PB, 128), lambda i: (0, i, 0)),
        pl.BlockSpec((PB, 128), lambda i: (i, 0)),
        pl.BlockSpec((PB, 128), lambda i: (i, 0)),
        pl.BlockSpec((1, 128), lambda i: (0, 0)),
        pl.BlockSpec((PB, 8), lambda i: (i, 0)),
        pl.BlockSpec((H, H), lambda i: (0, 0)),
        pl.BlockSpec((1, H), lambda i: (0, 0)),
        pl.BlockSpec((H, 1), lambda i: (0, 0)),
        pl.BlockSpec((1, 1), lambda i: (0, 0)),
    ],
    out_specs=pl.BlockSpec((G, 1), lambda i: (0, 0)),
    out_shape=jax.ShapeDtypeStruct((G, 1), _f32),
    scratch_shapes=[pltpu.VMEM((G, H + 1), _f32)],
)


def kernel(x, edge_index, batch, W1, b1, W2, b2, W3, b3, W4, b4, W5, b5,
           lin1_W, lin1_b, lin2_W, lin2_b):
    src = edge_index[0].astype(_i32)
    dst = edge_index[1].astype(_i32)
    src2d = jnp.concatenate([src, jnp.zeros((EPAD,), _i32)]).reshape(EROWS, LANE)
    dst2d = jnp.concatenate([dst, jnp.full((EPAD,), DUMMY, _i32)]).reshape(EROWS, LANE)

    x16 = jnp.concatenate([x, jnp.zeros((N, H - 4), _f32)], axis=1)
    x16 = jnp.concatenate([x16, jnp.zeros((NACC - N, H), _f32)], axis=0)
    x16_pk = x16.reshape(NP, 128)
    batch_p = jnp.concatenate([batch.astype(_i32), jnp.full((NACC - N,), G, _i32)])
    batch8 = batch_p.reshape(NP, 8)

    eye8 = jnp.eye(8, dtype=_f32)
    w1p = jnp.concatenate([W1, jnp.zeros((H - 4, H), _f32)], axis=0)

    dinvx = _prep_kernel(dst2d).reshape(NP, 128)
    z = _zinit(x16_pk, dinvx, jnp.kron(eye8, w1p))

    bs = [b1, b2, b3, b4]
    ws = [W2, W3, W4, W5]
    for b, w in zip(bs, ws):
        acc = _msg_kernel(z.reshape(NACC, H), src2d, dst2d)
        z = _layer(acc.reshape(NC, NP, 128), z, dinvx,
                   jnp.tile(b, 8).reshape(1, 128), jnp.kron(eye8, w))

    acc = _msg_kernel(z.reshape(NACC, H), src2d, dst2d)
    out = _pool(acc.reshape(NC, NP, 128), z, dinvx,
                jnp.tile(b5, 8).reshape(1, 128), batch8,
                lin1_W, lin1_b.reshape(1, H), lin2_W, lin2_b.reshape(1, 1))
    return out


# speedup vs baseline: 61.3260x; 1.0015x over previous
import jax
import jax.numpy as jnp
from jax import lax
from jax.experimental import pallas as pl
from jax.experimental.pallas import tpu as pltpu
from jax.experimental.pallas import tpu_sc as plsc

N = 100000
E = 3200000
H = 16
G = 64
SLOPE = (1.0 / 8.0 + 1.0 / 3.0) / 2.0

NC = 2
NS = 16
NW = NC * NS
LANE = 128

EROWS = 25088
EPAD = EROWS * LANE - E
RPT = EROWS // NW
CH = 4
NCHUNK = RPT // CH
RPT2 = EROWS // NS
NCHUNK2 = RPT2 // CH

NACC = 100096
RACC = NACC // NS
ZROWS = RACC // 8
DUMMY = N
NHALF = NACC // NC
NT = NACC // NW
NV = 196

NP = NACC * H // 128
PB = 368
PGRID = NP // PB

_f32 = jnp.float32
_i32 = jnp.int32

_sc_mesh = plsc.VectorSubcoreMesh(core_axis_name="c", subcore_axis_name="s")
_sc_params = pltpu.CompilerParams(use_tc_tiling_on_sc=False,
                                  needs_layout_passes=False)


def _prep_body(dst_hbm, out_hbm, dstv, onesv, dbuf, obuf, gb, degsh, ssem):
    c = lax.axis_index("c")
    s = lax.axis_index("s")

    def _zero(i, carry):
        obuf[pl.ds(i * 16, 16)] = jnp.zeros((16,), _f32)
        return carry

    lax.fori_loop(0, RACC // 16, _zero, 0)

    def _ones(i, carry):
        onesv[pl.ds(i * 16, 16)] = jnp.ones((16,), _f32)
        return carry

    lax.fori_loop(0, LANE // 16, _ones, 0)

    pltpu.sync_copy(obuf.at[pl.ds(0, RACC)], degsh.at[pl.ds(s * RACC, RACC)])
    plsc.subcore_barrier()

    def load_idx(slot, chunk):
        row0 = s * RPT2 + chunk * CH
        pltpu.sync_copy(dst_hbm.at[pl.ds(row0, CH)], dstv.at[slot])

    def fire_sc(slot):
        for j in range(CH):
            pltpu.async_copy(onesv, degsh.at[dstv.at[slot, j]], ssem,
                             add=True)

    def wait_sc(slot):
        for j in range(CH):
            pltpu.make_async_copy(onesv, degsh.at[dstv.at[slot, j]],
                                  ssem).wait()

    load_idx(0, 0)
    fire_sc(0)
    load_idx(1, 1)
    fire_sc(1)

    def _chunk(i, carry):
        p = lax.rem(i, 2)
        wait_sc(p)
        load_idx(p, i)
        fire_sc(p)
        return carry

    lax.fori_loop(2, NCHUNK2, _chunk, 0)
    wait_sc(NCHUNK2 % 2)
    wait_sc(1 - NCHUNK2 % 2)
    plsc.subcore_barrier()

    base = c * NHALF + s * NT
    pltpu.sync_copy(degsh.at[pl.ds(base, NT)], dbuf.at[pl.ds(0, NT)])

    def _rsqrt(v, carry):
        d = dbuf[pl.ds(v * 16, 16)] + 1.0
        ii = plsc.bitcast(d, _i32)
        gi = jnp.full((16,), 0x5F3759DF, _i32) - (ii >> 1)
        g = plsc.bitcast(gi, _f32)
        hx = 0.5 * d
        g = g * (1.5 - hx * g * g)
        g = g * (1.5 - hx * g * g)
        g = g * (1.5 - hx * g * g)
        gb[...] = g
        for a in range(16):
            vec = plsc.load_gather(gb, [jnp.full((16,), a, _i32)])
            obuf[pl.ds(v * 256 + a * 16, 16)] = vec
        return carry

    lax.fori_loop(0, NV, _rsqrt, 0)
    pltpu.sync_copy(obuf.at[pl.ds(0, NT * 16)],
                    out_hbm.at[pl.ds(base * 16, NT * 16)])


_prep_kernel = pl.kernel(
    _prep_body,
    out_type=jax.ShapeDtypeStruct((NACC * H,), _f32),
    mesh=_sc_mesh,
    scratch_types=[
        pltpu.VMEM((2, CH, LANE), _i32),
        pltpu.VMEM((LANE,), _f32),
        pltpu.VMEM((NT + 8,), _f32),
        pltpu.VMEM((NT * 16,), _f32),
        pltpu.VMEM((16,), _f32),
        pltpu.VMEM_SHARED((NACC,), _f32),
        pltpu.SemaphoreType.DMA,
    ],
    compiler_params=_sc_params,
)


def _msg_body(z_hbm, src_hbm, dst_hbm, out_hbm, srcv, dstv, rowsv, zbuf,
              accsh, gsem, ssem):
    c = lax.axis_index("c")
    s = lax.axis_index("s")
    wid = s * NC + c

    def _zero(i, carry):
        zbuf[i] = jnp.zeros((16,), _f32)
        return carry

    lax.fori_loop(0, ZROWS, _zero, 0)
    for k in range(RACC // ZROWS):
        pltpu.sync_copy(zbuf, accsh.at[pl.ds(s * RACC + k * ZROWS, ZROWS)])
    plsc.subcore_barrier()

    def load_idx(slot, chunk):
        row0 = wid * RPT + chunk * CH
        pltpu.sync_copy(src_hbm.at[pl.ds(row0, CH)], srcv.at[slot])
        pltpu.sync_copy(dst_hbm.at[pl.ds(row0, CH)], dstv.at[slot])

    def fire_g(slot):
        for j in range(CH):
            pltpu.async_copy(z_hbm.at[srcv.at[slot, j]], rowsv.at[slot, j],
                             gsem)

    def wait_g(slot):
        for j in range(CH):
            pltpu.make_async_copy(z_hbm.at[srcv.at[slot, j]],
                                  rowsv.at[slot, j], gsem).wait()

    def fire_s(slot):
        for j in range(CH):
            pltpu.async_copy(rowsv.at[slot, j], accsh.at[dstv.at[slot, j]],
                             ssem, add=True)

    def wait_s(slot):
        for j in range(CH):
            pltpu.make_async_copy(rowsv.at[slot, j],
                                  accsh.at[dstv.at[slot, j]], ssem).wait()

    load_idx(0, 0)
    fire_g(0)
    load_idx(1, 1)
    wait_g(0)
    fire_s(0)
    fire_g(1)

    def _chunk(i, carry):
        p = lax.rem(i, 2)
        q = 1 - p
        wait_s(q)
        load_idx(q, i + 1)
        wait_g(p)
        fire_s(p)
        fire_g(q)
        return carry

    lax.fori_loop(1, NCHUNK - 1, _chunk, 0)
    pl_last = (NCHUNK - 1) % 2
    wait_s(1 - pl_last)
    wait_g(pl_last)
    fire_s(pl_last)
    wait_s(pl_last)
    plsc.subcore_barrier()
    pltpu.sync_copy(accsh.at[pl.ds(s * RACC, RACC)],
                    out_hbm.at[c, pl.ds(s * RACC, RACC)])


_msg_kernel = pl.kernel(
    _msg_body,
    out_type=jax.ShapeDtypeStruct((NC, NACC, H), _f32),
    mesh=_sc_mesh,
    scratch_types=[
        pltpu.VMEM((2, CH, LANE), _i32),
        pltpu.VMEM((2, CH, LANE), _i32),
        pltpu.VMEM((2, CH, LANE, H), _f32),
        pltpu.VMEM((ZROWS, H), _f32),
        pltpu.VMEM_SHARED((NACC, H), _f32),
        pltpu.SemaphoreType.DMA,
        pltpu.SemaphoreType.DMA,
    ],
    compiler_params=_sc_params,
)


def _zinit_body(x_ref, dinvx_ref, w_ref, z_ref):
    y = jnp.dot(x_ref[...], w_ref[...], preferred_element_type=_f32)
    z_ref[...] = y * dinvx_ref[...]


_zinit = pl.pallas_call(
    _zinit_body,
    grid=(PGRID,),
    in_specs=[
        pl.BlockSpec((PB, 128), lambda i: (i, 0)),
        pl.BlockSpec((PB, 128), lambda i: (i, 0)),
        pl.BlockSpec((128, 128), lambda i: (0, 0)),
    ],
    out_specs=pl.BlockSpec((PB, 128), lambda i: (i, 0)),
    out_shape=jax.ShapeDtypeStruct((NP, 128), _f32),
)


def _layer_body(acc_ref, z_ref, dinvx_ref, b8_ref, w8_ref, out_ref):
    dinvx = dinvx_ref[...]
    pre = (acc_ref[0] + acc_ref[1] + z_ref[...]) * dinvx + b8_ref[...]
    h = jnp.maximum(pre, 0.0)
    out_ref[...] = jnp.dot(h, w8_ref[...],
                           preferred_element_type=_f32) * dinvx


_layer = pl.pallas_call(
    _layer_body,
    grid=(PGRID,),
    in_specs=[
        pl.BlockSpec((NC, PB, 128), lambda i: (0, i, 0)),
        pl.BlockSpec((PB, 128), lambda i: (i, 0)),
        pl.BlockSpec((PB, 128), lambda i: (i, 0)),
        pl.BlockSpec((1, 128), lambda i: (0, 0)),
        pl.BlockSpec((128, 128), lambda i: (0, 0)),
    ],
    out_specs=pl.BlockSpec((PB, 128), lambda i: (i, 0)),
    out_shape=jax.ShapeDtypeStruct((NP, 128), _f32),
)


def _pool_body(acc_ref, z_ref, dinvx_ref, b8_ref, batch_ref, l1w_ref,
               l1b_ref, l2w_ref, l2b_ref, out_ref, sums_ref):
    i = pl.program_id(0)

    @pl.when(i == 0)
    def _():
        sums_ref[...] = jnp.zeros_like(sums_ref)

    pre = (acc_ref[0] + acc_ref[1] + z_ref[...]) * dinvx_ref[...]
    h_pk = jnp.maximum(pre + b8_ref[...], 0.0)
    iota_g = lax.broadcasted_iota(_i32, (1, G), 1)
    acc = jnp.zeros((G, H + 1), _f32)
    for a in range(8):
        ha = h_pk[:, a * H:(a + 1) * H]
        hha = jnp.concatenate([ha, jnp.ones((PB, 1), _f32)], axis=1)
        sega = (batch_ref[:, a:a + 1] == iota_g).astype(_f32)
        acc = acc + lax.dot_general(sega, hha, (((0,), (0,)), ((), ())),
                                    preferred_element_type=_f32)
    sums_ref[...] += acc

    @pl.when(i == PGRID - 1)
    def _():
        tot = sums_ref[...]
        mean = tot[:, :H] / jnp.maximum(tot[:, H:H + 1], 1.0)
        g = jnp.dot(mean, l1w_ref[...], preferred_element_type=_f32)
        g = g + l1b_ref[...]
        g = jnp.where(g >= 0, g, SLOPE * g)
        o = jnp.dot(g, l2w_ref[...], preferred_element_type=_f32)
        o = o + l2b_ref[...]
        out_ref[...] = jnp.where(o >= 0, o, SLOPE * o)


_pool = pl.pallas_call(
    _pool_body,
    grid=(PGRID,),
    in_specs=[
        pl.BlockSpec((NC, PB, 128), lambda i: (0, i, 0)),
        pl.BlockSpec((PB, 128), lambda i: (i, 0)),
        pl.BlockSpec((PB, 128), lambda i: (i, 0)),
        pl.BlockSpec((1, 128), lambda i: (0, 0)),
        pl.BlockSpec((PB, 8), lambda i: (i, 0)),
        pl.BlockSpec((H, H), lambda i: (0, 0)),
        pl.BlockSpec((1, H), lambda i: (0, 0)),
        pl.BlockSpec((H, 1), lambda i: (0, 0)),
        pl.BlockSpec((1, 1), lambda i: (0, 0)),
    ],
    out_specs=pl.BlockSpec((G, 1), lambda i: (0, 0)),
    out_shape=jax.ShapeDtypeStruct((G, 1), _f32),
    scratch_shapes=[pltpu.VMEM((G, H + 1), _f32)],
)


def kernel(x, edge_index, batch, W1, b1, W2, b2, W3, b3, W4, b4, W5, b5,
           lin1_W, lin1_b, lin2_W, lin2_b):
    src = edge_index[0].astype(_i32)
    dst = edge_index[1].astype(_i32)
    src2d = jnp.pad(src, (0, EPAD)).reshape(EROWS, LANE)
    dst2d = jnp.pad(dst, (0, EPAD), constant_values=DUMMY).reshape(EROWS, LANE)

    x16_pk = jnp.pad(x, ((0, NACC - N), (0, H - 4))).reshape(NP, 128)
    batch8 = jnp.pad(batch.astype(_i32), (0, NACC - N),
                     constant_values=G).reshape(NP, 8)

    eye8 = jnp.eye(8, dtype=_f32)
    w1p = jnp.concatenate([W1, jnp.zeros((H - 4, H), _f32)], axis=0)

    dinvx = _prep_kernel(dst2d).reshape(NP, 128)
    z = _zinit(x16_pk, dinvx, jnp.kron(eye8, w1p))

    bs = [b1, b2, b3, b4]
    ws = [W2, W3, W4, W5]
    for b, w in zip(bs, ws):
        acc = _msg_kernel(z.reshape(NACC, H), src2d, dst2d)
        z = _layer(acc.reshape(NC, NP, 128), z, dinvx,
                   jnp.tile(b, 8).reshape(1, 128), jnp.kron(eye8, w))

    acc = _msg_kernel(z.reshape(NACC, H), src2d, dst2d)
    out = _pool(acc.reshape(NC, NP, 128), z, dinvx,
                jnp.tile(b5, 8).reshape(1, 128), batch8,
                lin1_W, lin1_b.reshape(1, H), lin2_W, lin2_b.reshape(1, 1))
    return out
